# Initial kernel scaffold; baseline (speedup 1.0000x reference)
import jax
import jax.numpy as jnp
from jax.experimental import pallas as pl


def kernel(x, token_table, path_table, attn_param, W, b):
    raise NotImplementedError("write your pallas kernel here")


# speedup vs baseline: 4.5846x; 4.5846x over previous
import functools

import jax
import jax.numpy as jnp
from jax import lax
from jax.experimental import pallas as pl
from jax.experimental.pallas import tpu as pltpu
from jax.experimental.pallas import tpu_sc as plsc

_NC = 2
_NS = 16
_NW = _NC * _NS
_CHUNK = 128


def _sc_gather(token_table, path_table, i0, i1, i2):
    (BL,) = i0.shape
    V, D = token_table.shape
    rows_per_w = BL // _NW
    nchunk = rows_per_w // _CHUNK
    assert rows_per_w * _NW == BL and nchunk * _CHUNK == rows_per_w

    mesh = plsc.VectorSubcoreMesh(
        core_axis_name="c", subcore_axis_name="s",
        num_cores=_NC, num_subcores=_NS)
    out_t = jax.ShapeDtypeStruct((BL, D), jnp.float32)

    @functools.partial(
        pl.kernel,
        out_type=(out_t, out_t, out_t),
        mesh=mesh,
        scratch_types=[
            pltpu.VMEM((_CHUNK,), jnp.int32),
            pltpu.VMEM((_CHUNK,), jnp.int32),
            pltpu.VMEM((_CHUNK,), jnp.int32),
            pltpu.VMEM((_CHUNK, D), jnp.float32),
            pltpu.VMEM((_CHUNK, D), jnp.float32),
            pltpu.VMEM((_CHUNK, D), jnp.float32),
            pltpu.SemaphoreType.DMA,
            pltpu.SemaphoreType.DMA,
            pltpu.SemaphoreType.DMA,
        ],
    )
    def gather_kernel(tok_hbm, path_hbm, i0_hbm, i1_hbm, i2_hbm,
                      o0_hbm, o1_hbm, o2_hbm,
                      x0_v, x1_v, x2_v, r0_v, r1_v, r2_v, s0, s1, s2):
        wid = lax.axis_index("s") * _NC + lax.axis_index("c")
        w_base = wid * rows_per_w

        def body(k, carry):
            base = w_base + k * _CHUNK
            pltpu.sync_copy(i0_hbm.at[pl.ds(base, _CHUNK)], x0_v)
            pltpu.sync_copy(i1_hbm.at[pl.ds(base, _CHUNK)], x1_v)
            pltpu.sync_copy(i2_hbm.at[pl.ds(base, _CHUNK)], x2_v)
            c0 = pltpu.async_copy(tok_hbm.at[x0_v], r0_v, s0)
            c1 = pltpu.async_copy(path_hbm.at[x1_v], r1_v, s1)
            c2 = pltpu.async_copy(tok_hbm.at[x2_v], r2_v, s2)
            c0.wait()
            c1.wait()
            c2.wait()
            pltpu.sync_copy(r0_v, o0_hbm.at[pl.ds(base, _CHUNK)])
            pltpu.sync_copy(r1_v, o1_hbm.at[pl.ds(base, _CHUNK)])
            pltpu.sync_copy(r2_v, o2_hbm.at[pl.ds(base, _CHUNK)])
            return carry

        lax.fori_loop(0, nchunk, body, 0)

    return gather_kernel(token_table, path_table, i0, i1, i2)


def _dense_body(L, D, tl_ref, pp_ref, tr_ref, w_ref, b_ref, ap_ref, o_ref):
    Bt = tl_ref.shape[0]
    f32 = jnp.float32
    tl = tl_ref[...].reshape(Bt * L, D)
    pp = pp_ref[...].reshape(Bt * L, D)
    tr = tr_ref[...].reshape(Bt * L, D)
    ct = jnp.tanh(
        jnp.dot(tl, w_ref[0:D, :], preferred_element_type=f32)
        + jnp.dot(pp, w_ref[D:2 * D, :], preferred_element_type=f32)
        + jnp.dot(tr, w_ref[2 * D:3 * D, :], preferred_element_type=f32)
        + b_ref[...]
    )
    ct3 = ct.reshape(Bt, L, D)
    a = jnp.sum(ct3 * ap_ref[...].reshape(1, 1, D), axis=2, keepdims=True)
    m = jnp.max(a, axis=1, keepdims=True)
    e = jnp.exp(a - m)
    p = e / jnp.sum(e, axis=1, keepdims=True)
    o_ref[...] = jnp.sum(ct3 * p, axis=1)


def _tc_dense(tl, pp, tr, W, b2, ap2, Bt=8):
    B, L, D = tl.shape
    grid = (B // Bt,)
    blk = pl.BlockSpec((Bt, L, D), lambda i: (i, 0, 0))
    return pl.pallas_call(
        functools.partial(_dense_body, L, D),
        grid=grid,
        in_specs=[
            blk, blk, blk,
            pl.BlockSpec((3 * D, D), lambda i: (0, 0)),
            pl.BlockSpec((1, D), lambda i: (0, 0)),
            pl.BlockSpec((1, D), lambda i: (0, 0)),
        ],
        out_specs=pl.BlockSpec((Bt, D), lambda i: (i, 0)),
        out_shape=jax.ShapeDtypeStruct((B, D), jnp.float32),
    )(tl, pp, tr, W, b2, ap2)


def kernel(x, token_table, path_table, attn_param, W, b):
    B, L, _ = x.shape
    V, D = token_table.shape
    BL = B * L
    xf = x.reshape(BL, 3)
    i0 = xf[:, 0]
    i1 = xf[:, 1]
    i2 = xf[:, 2]
    o0, o1, o2 = _sc_gather(token_table, path_table, i0, i1, i2)
    tl = o0.reshape(B, L, D)
    pp = o1.reshape(B, L, D)
    tr = o2.reshape(B, L, D)
    b2 = b.reshape(1, D)
    ap2 = attn_param.reshape(1, D)
    return _tc_dense(tl, pp, tr, W, b2, ap2)


# speedup vs baseline: 5.0038x; 1.0914x over previous
import functools

import jax
import jax.numpy as jnp
from jax import lax
from jax.experimental import pallas as pl
from jax.experimental.pallas import tpu as pltpu
from jax.experimental.pallas import tpu_sc as plsc

_NC = 2
_NS = 16
_NW = _NC * _NS
_CHUNK = 128


def _transform_body(D, tok_ref, path_ref, w_ref, t1_ref, p2_ref, t3_ref):
    f32 = jnp.float32
    tok = tok_ref[...]
    t1_ref[...] = jnp.dot(tok, w_ref[0:D, :], preferred_element_type=f32)
    p2_ref[...] = jnp.dot(path_ref[...], w_ref[D:2 * D, :],
                          preferred_element_type=f32)
    t3_ref[...] = jnp.dot(tok, w_ref[2 * D:3 * D, :],
                          preferred_element_type=f32)


def _tc_transform(token_table, path_table, W, Vt=800):
    V, D = token_table.shape
    assert V % Vt == 0
    grid = (V // Vt,)
    tblk = pl.BlockSpec((Vt, D), lambda i: (i, 0))
    out_t = jax.ShapeDtypeStruct((V, D), jnp.float32)
    return pl.pallas_call(
        functools.partial(_transform_body, D),
        grid=grid,
        in_specs=[tblk, tblk, pl.BlockSpec((3 * D, D), lambda i: (0, 0))],
        out_specs=[tblk, tblk, tblk],
        out_shape=[out_t, out_t, out_t],
    )(token_table, path_table, W)


def _sc_gather_sum(t1, p2, t3, i0, i1, i2):
    (BL,) = i0.shape
    V, D = t1.shape
    rows_per_w = BL // _NW
    nchunk = rows_per_w // _CHUNK
    npair = nchunk // 2
    assert rows_per_w * _NW == BL and npair * 2 * _CHUNK == rows_per_w
    ngrp = D // 16

    mesh = plsc.VectorSubcoreMesh(
        core_axis_name="c", subcore_axis_name="s",
        num_cores=_NC, num_subcores=_NS)

    idx_t = pltpu.VMEM((_CHUNK,), jnp.int32)
    row_t = pltpu.VMEM((_CHUNK, D), jnp.float32)

    @functools.partial(
        pl.kernel,
        out_type=jax.ShapeDtypeStruct((BL, D), jnp.float32),
        mesh=mesh,
        scratch_types=[
            [idx_t] * 3, [idx_t] * 3,
            [row_t] * 3, [row_t] * 3,
            pltpu.SemaphoreType.DMA, pltpu.SemaphoreType.DMA,
        ],
    )
    def gather_kernel(t1_hbm, p2_hbm, t3_hbm, i0_hbm, i1_hbm, i2_hbm,
                      o_hbm, idx_a, idx_b, rows_a, rows_b, sem_a, sem_b):
        wid = lax.axis_index("s") * _NC + lax.axis_index("c")
        w_base = wid * rows_per_w
        tabs = (t1_hbm, p2_hbm, t3_hbm)
        idxs = (i0_hbm, i1_hbm, i2_hbm)

        def stage(chunk_no, idx_v, rows_v, sem):
            base = w_base + chunk_no * _CHUNK
            for t in range(3):
                pltpu.sync_copy(idxs[t].at[pl.ds(base, _CHUNK)], idx_v[t])
            for t in range(3):
                pltpu.async_copy(tabs[t].at[idx_v[t]], rows_v[t], sem)

        def drain(rows_v, sem):
            for t in range(3):
                pltpu.make_async_copy(tabs[t].at[pl.ds(0, _CHUNK)], rows_v[t],
                                      sem).wait()

        def add_store(chunk_no, rows_v):
            base = w_base + chunk_no * _CHUNK
            r0, r1, r2 = rows_v

            def row_body(r, carry):
                for g in range(ngrp):
                    sl = (r, pl.ds(g * 16, 16))
                    r0[sl] = r0[sl] + r1[sl] + r2[sl]
                return carry

            lax.fori_loop(0, _CHUNK, row_body, 0)
            pltpu.sync_copy(r0, o_hbm.at[pl.ds(base, _CHUNK)])

        stage(0, idx_a, rows_a, sem_a)

        def pair_body(j, carry):
            stage(2 * j + 1, idx_b, rows_b, sem_b)
            drain(rows_a, sem_a)
            add_store(2 * j, rows_a)

            @pl.when(j + 1 < npair)
            def _():
                stage(2 * j + 2, idx_a, rows_a, sem_a)

            drain(rows_b, sem_b)
            add_store(2 * j + 1, rows_b)
            return carry

        lax.fori_loop(0, npair, pair_body, 0)

    return gather_kernel(t1, p2, t3, i0, i1, i2)


def _tail_body(L, D, s_ref, b_ref, ap_ref, o_ref):
    Bt = s_ref.shape[0]
    ct = jnp.tanh(s_ref[...] + b_ref[...].reshape(1, 1, D))
    a = jnp.sum(ct * ap_ref[...].reshape(1, 1, D), axis=2, keepdims=True)
    m = jnp.max(a, axis=1, keepdims=True)
    e = jnp.exp(a - m)
    p = e / jnp.sum(e, axis=1, keepdims=True)
    o_ref[...] = jnp.sum(ct * p, axis=1)


def _tc_tail(s3, b2, ap2, Bt=8):
    B, L, D = s3.shape
    grid = (B // Bt,)
    return pl.pallas_call(
        functools.partial(_tail_body, L, D),
        grid=grid,
        in_specs=[
            pl.BlockSpec((Bt, L, D), lambda i: (i, 0, 0)),
            pl.BlockSpec((1, D), lambda i: (0, 0)),
            pl.BlockSpec((1, D), lambda i: (0, 0)),
        ],
        out_specs=pl.BlockSpec((Bt, D), lambda i: (i, 0)),
        out_shape=jax.ShapeDtypeStruct((B, D), jnp.float32),
    )(s3, b2, ap2)


def kernel(x, token_table, path_table, attn_param, W, b):
    B, L, _ = x.shape
    V, D = token_table.shape
    BL = B * L
    xf = x.reshape(BL, 3)
    i0 = xf[:, 0]
    i1 = xf[:, 1]
    i2 = xf[:, 2]
    t1, p2, t3 = _tc_transform(token_table, path_table, W)
    s = _sc_gather_sum(t1, p2, t3, i0, i1, i2)
    return _tc_tail(s.reshape(B, L, D), b.reshape(1, D),
                    attn_param.reshape(1, D))
